# Initial kernel scaffold; baseline (speedup 1.0000x reference)
import jax
import jax.numpy as jnp
from jax.experimental import pallas as pl


def kernel(x, edge_index, W1, b1, A1, Ab1, W2, b2, A2, Ab2):
    raise NotImplementedError("write your pallas kernel here")


# speedup vs baseline: 5.2942x; 5.2942x over previous
import functools

import jax
import jax.numpy as jnp
from jax import lax
from jax.experimental import pallas as pl
from jax.experimental.pallas import tpu as pltpu
from jax.experimental.pallas import tpu_sc as plsc

N = 10000
E = 160000
F_IN = 256
H = 16
C = 16

NC = 2
NS = 16
NW = NC * NS
CH = 128
CPW = 40
E_PAD = NW * CPW * CH
N_PAD = 10112
WIDTH = 128


def _prologue1_body(x_ref, w_ref, b_ref, a_ref, ab_ref, out_ref):
    xn = jnp.dot(x_ref[...], w_ref[...], preferred_element_type=jnp.float32)
    xn = xn + b_ref[...]
    ar = jnp.dot(xn, a_ref[0:H, :], preferred_element_type=jnp.float32)
    u = (ar + ab_ref[...]) * xn
    z = jnp.zeros((xn.shape[0], WIDTH - 3 * H), jnp.float32)
    out_ref[...] = jnp.concatenate([u, xn, jnp.ones_like(xn), z], axis=1)


def _mid_body(acc_ref, t1_ref, a1_ref, ab1_ref, w2_ref, b2_ref, a2_ref,
              ab2_ref, t2_ref, cnt_ref):
    S = acc_ref[0] + acc_ref[1]
    S1 = S[:, 0:H]
    S2 = S[:, H:2 * H]
    deg = S[:, 2 * H:3 * H]
    xn1 = t1_ref[:, H:2 * H]
    ar1 = jnp.dot(xn1, a1_ref[0:H, :], preferred_element_type=jnp.float32)
    ac1 = jnp.dot(xn1, a1_ref[H:2 * H, :], preferred_element_type=jnp.float32)
    self1 = (ar1 + ac1 + ab1_ref[...]) * xn1
    cnt = deg + 1.0
    h = jnp.maximum((S1 + ac1 * S2 + self1) / cnt, 0.0)
    xn2 = jnp.dot(h, w2_ref[...], preferred_element_type=jnp.float32)
    xn2 = xn2 + b2_ref[...]
    ar2 = jnp.dot(xn2, a2_ref[0:C, :], preferred_element_type=jnp.float32)
    u2 = (ar2 + ab2_ref[...]) * xn2
    z = jnp.zeros((xn2.shape[0], WIDTH - 2 * C), jnp.float32)
    t2_ref[...] = jnp.concatenate([u2, xn2, z], axis=1)
    cnt_ref[...] = cnt


def _final_body(acc_ref, t2_ref, cnt_ref, a2_ref, ab2_ref, out_ref):
    S = acc_ref[0] + acc_ref[1]
    S1 = S[:, 0:C]
    S2 = S[:, C:2 * C]
    xn2 = t2_ref[:, C:2 * C]
    ar2 = jnp.dot(xn2, a2_ref[0:C, :], preferred_element_type=jnp.float32)
    ac2 = jnp.dot(xn2, a2_ref[C:2 * C, :], preferred_element_type=jnp.float32)
    self2 = (ar2 + ac2 + ab2_ref[...]) * xn2
    o = (S1 + ac2 * S2 + self2) / cnt_ref[...]
    m = jnp.max(o, axis=1, keepdims=True)
    om = o - m
    lse = jnp.log(jnp.sum(jnp.exp(om), axis=1, keepdims=True))
    out_ref[...] = om - lse


def _make_sc_pass(width):
    mesh = plsc.VectorSubcoreMesh(core_axis_name="c", subcore_axis_name="s",
                                  num_cores=NC, num_subcores=NS)
    rpt = N_PAD // NS

    @functools.partial(
        pl.kernel,
        out_type=jax.ShapeDtypeStruct((NC, N_PAD, width), jnp.float32),
        mesh=mesh,
        scratch_types=[
            pltpu.VMEM((CPW, CH), jnp.int32),
            pltpu.VMEM((CPW, CH), jnp.int32),
            pltpu.VMEM((CH, width), jnp.float32),
            pltpu.VMEM_SHARED((N_PAD, width), jnp.float32),
            pltpu.SemaphoreType.DMA,
        ],
    )
    def sc_pass(table, rowi, coli, zeros, out, idxr, idxc, rows, acc, sem):
        c = lax.axis_index("c")
        s = lax.axis_index("s")
        wid = c * NS + s
        pltpu.sync_copy(zeros.at[pl.ds(s * rpt, rpt)],
                        acc.at[pl.ds(s * rpt, rpt)])
        pltpu.sync_copy(rowi.at[pl.ds(wid * CPW, CPW)], idxr)
        pltpu.sync_copy(coli.at[pl.ds(wid * CPW, CPW)], idxc)
        plsc.subcore_barrier()

        def body(j, carry):
            pltpu.async_copy(table.at[idxr.at[j]], rows, sem).wait()
            pltpu.sync_copy(rows, acc.at[idxc.at[j]], add=True)
            return carry

        lax.fori_loop(0, CPW, body, 0)
        plsc.subcore_barrier()
        pltpu.sync_copy(acc.at[pl.ds(s * rpt, rpt)],
                        out.at[c, pl.ds(s * rpt, rpt)])

    return sc_pass


_make_sc_pass = functools.lru_cache(maxsize=None)(_make_sc_pass)


@jax.jit
def kernel(x, edge_index, W1, b1, A1, Ab1, W2, b2, A2, Ab2):
    f32 = jnp.float32
    pad = jnp.full((E_PAD - E,), N, dtype=jnp.int32)
    rowi = jnp.concatenate([edge_index[0], pad]).reshape(NW * CPW, CH)
    coli = jnp.concatenate([edge_index[1], pad]).reshape(NW * CPW, CH)

    table1 = pl.pallas_call(
        _prologue1_body,
        out_shape=jax.ShapeDtypeStruct((N, WIDTH), f32),
    )(x, W1, b1.reshape(1, H), A1, Ab1.reshape(1, 1))
    table1 = jnp.concatenate(
        [table1, jnp.zeros((N_PAD - N, WIDTH), f32)], axis=0)

    zeros_t = jnp.zeros((N_PAD, WIDTH), f32)
    acc1 = _make_sc_pass(WIDTH)(table1, rowi, coli, zeros_t)

    table2, cnt = pl.pallas_call(
        _mid_body,
        out_shape=(
            jax.ShapeDtypeStruct((N_PAD, WIDTH), f32),
            jax.ShapeDtypeStruct((N_PAD, C), f32),
        ),
    )(acc1, table1, A1, Ab1.reshape(1, 1), W2, b2.reshape(1, C), A2,
      Ab2.reshape(1, 1))

    acc2 = _make_sc_pass(WIDTH)(table2, rowi, coli, zeros_t)

    out = pl.pallas_call(
        _final_body,
        out_shape=jax.ShapeDtypeStruct((N_PAD, C), f32),
    )(acc2, table2, cnt, A2, Ab2.reshape(1, 1))
    return out[:N]


# speedup vs baseline: 5.7975x; 1.0951x over previous
import functools

import jax
import jax.numpy as jnp
from jax import lax
from jax.experimental import pallas as pl
from jax.experimental.pallas import tpu as pltpu
from jax.experimental.pallas import tpu_sc as plsc

N = 10000
E = 160000
F_IN = 256
H = 16
C = 16

NC = 2
NS = 16
NW = NC * NS
CH = 128
CPW = 40
E_PAD = NW * CPW * CH
N_PAD = 10112
WIDTH = 128


def _prologue1_body(x_ref, w_ref, b_ref, a_ref, ab_ref, out_ref):
    xn = jnp.dot(x_ref[...], w_ref[...], preferred_element_type=jnp.float32)
    xn = xn + b_ref[...]
    ar = jnp.dot(xn, a_ref[0:H, :], preferred_element_type=jnp.float32)
    u = (ar + ab_ref[...]) * xn
    z = jnp.zeros((xn.shape[0], WIDTH - 3 * H), jnp.float32)
    out_ref[...] = jnp.concatenate([u, xn, jnp.ones_like(xn), z], axis=1)


def _mid_body(acc_ref, t1_ref, a1_ref, ab1_ref, w2_ref, b2_ref, a2_ref,
              ab2_ref, t2_ref, cnt_ref):
    S = acc_ref[0] + acc_ref[1]
    S1 = S[:, 0:H]
    S2 = S[:, H:2 * H]
    deg = S[:, 2 * H:3 * H]
    xn1 = t1_ref[:, H:2 * H]
    ar1 = jnp.dot(xn1, a1_ref[0:H, :], preferred_element_type=jnp.float32)
    ac1 = jnp.dot(xn1, a1_ref[H:2 * H, :], preferred_element_type=jnp.float32)
    self1 = (ar1 + ac1 + ab1_ref[...]) * xn1
    cnt = deg + 1.0
    h = jnp.maximum((S1 + ac1 * S2 + self1) / cnt, 0.0)
    xn2 = jnp.dot(h, w2_ref[...], preferred_element_type=jnp.float32)
    xn2 = xn2 + b2_ref[...]
    ar2 = jnp.dot(xn2, a2_ref[0:C, :], preferred_element_type=jnp.float32)
    u2 = (ar2 + ab2_ref[...]) * xn2
    z = jnp.zeros((xn2.shape[0], WIDTH - 2 * C), jnp.float32)
    t2_ref[...] = jnp.concatenate([u2, xn2, z], axis=1)
    cnt_ref[...] = cnt


def _final_body(acc_ref, t2_ref, cnt_ref, a2_ref, ab2_ref, out_ref):
    S = acc_ref[0] + acc_ref[1]
    S1 = S[:, 0:C]
    S2 = S[:, C:2 * C]
    xn2 = t2_ref[:, C:2 * C]
    ar2 = jnp.dot(xn2, a2_ref[0:C, :], preferred_element_type=jnp.float32)
    ac2 = jnp.dot(xn2, a2_ref[C:2 * C, :], preferred_element_type=jnp.float32)
    self2 = (ar2 + ac2 + ab2_ref[...]) * xn2
    o = (S1 + ac2 * S2 + self2) / cnt_ref[...]
    m = jnp.max(o, axis=1, keepdims=True)
    om = o - m
    lse = jnp.log(jnp.sum(jnp.exp(om), axis=1, keepdims=True))
    out_ref[...] = om - lse


def _make_sc_pass(width):
    mesh = plsc.VectorSubcoreMesh(core_axis_name="c", subcore_axis_name="s",
                                  num_cores=NC, num_subcores=NS)
    rpt = N_PAD // NS

    @functools.partial(
        pl.kernel,
        out_type=jax.ShapeDtypeStruct((NC, N_PAD, width), jnp.float32),
        mesh=mesh,
        scratch_types=[
            pltpu.VMEM((CPW, CH), jnp.int32),
            pltpu.VMEM((CPW, CH), jnp.int32),
            pltpu.VMEM((CH, width), jnp.float32),
            pltpu.VMEM((CH, width), jnp.float32),
            pltpu.VMEM_SHARED((N_PAD, width), jnp.float32),
            pltpu.SemaphoreType.DMA,
            pltpu.SemaphoreType.DMA,
        ],
    )
    def sc_pass(table, rowi, coli, zeros, out, idxr, idxc, rows0, rows1,
                acc, sem0, sem1):
        c = lax.axis_index("c")
        s = lax.axis_index("s")
        wid = c * NS + s
        pltpu.sync_copy(zeros.at[pl.ds(s * rpt, rpt)],
                        acc.at[pl.ds(s * rpt, rpt)])
        pltpu.sync_copy(rowi.at[pl.ds(wid * CPW, CPW)], idxr)
        pltpu.sync_copy(coli.at[pl.ds(wid * CPW, CPW)], idxc)
        plsc.subcore_barrier()

        pltpu.async_copy(table.at[idxr.at[0]], rows0, sem0)

        def body(k, carry):
            j0 = 2 * k
            pltpu.async_copy(table.at[idxr.at[j0 + 1]], rows1, sem1)
            pltpu.make_async_copy(table.at[idxr.at[j0]], rows0, sem0).wait()
            pltpu.sync_copy(rows0, acc.at[idxc.at[j0]], add=True)

            @pl.when(j0 + 2 < CPW)
            def _():
                pltpu.async_copy(table.at[idxr.at[j0 + 2]], rows0, sem0)

            pltpu.make_async_copy(
                table.at[idxr.at[j0 + 1]], rows1, sem1).wait()
            pltpu.sync_copy(rows1, acc.at[idxc.at[j0 + 1]], add=True)
            return carry

        lax.fori_loop(0, CPW // 2, body, 0)
        plsc.subcore_barrier()
        pltpu.sync_copy(acc.at[pl.ds(s * rpt, rpt)],
                        out.at[c, pl.ds(s * rpt, rpt)])

    return sc_pass


_make_sc_pass = functools.lru_cache(maxsize=None)(_make_sc_pass)


@jax.jit
def kernel(x, edge_index, W1, b1, A1, Ab1, W2, b2, A2, Ab2):
    f32 = jnp.float32
    pad = jnp.full((E_PAD - E,), N, dtype=jnp.int32)
    rowi = jnp.concatenate([edge_index[0], pad]).reshape(NW * CPW, CH)
    coli = jnp.concatenate([edge_index[1], pad]).reshape(NW * CPW, CH)

    table1 = pl.pallas_call(
        _prologue1_body,
        out_shape=jax.ShapeDtypeStruct((N, WIDTH), f32),
    )(x, W1, b1.reshape(1, H), A1, Ab1.reshape(1, 1))
    table1 = jnp.concatenate(
        [table1, jnp.zeros((N_PAD - N, WIDTH), f32)], axis=0)

    zeros_t = jnp.zeros((N_PAD, WIDTH), f32)
    acc1 = _make_sc_pass(WIDTH)(table1, rowi, coli, zeros_t)

    table2, cnt = pl.pallas_call(
        _mid_body,
        out_shape=(
            jax.ShapeDtypeStruct((N_PAD, WIDTH), f32),
            jax.ShapeDtypeStruct((N_PAD, C), f32),
        ),
    )(acc1, table1, A1, Ab1.reshape(1, 1), W2, b2.reshape(1, C), A2,
      Ab2.reshape(1, 1))

    acc2 = _make_sc_pass(WIDTH)(table2, rowi, coli, zeros_t)

    out = pl.pallas_call(
        _final_body,
        out_shape=jax.ShapeDtypeStruct((N_PAD, C), f32),
    )(acc2, table2, cnt, A2, Ab2.reshape(1, 1))
    return out[:N]


# speedup vs baseline: 6.1967x; 1.0689x over previous
import functools

import jax
import jax.numpy as jnp
from jax import lax
from jax.experimental import pallas as pl
from jax.experimental.pallas import tpu as pltpu
from jax.experimental.pallas import tpu_sc as plsc

N = 10000
E = 160000
F_IN = 256
H = 16
C = 16

NC = 2
NS = 16
NW = NC * NS
CH = 128
CPW0 = 16
CPW1 = 64
N_CHUNKS = NS * (CPW0 + CPW1)
E_PAD = N_CHUNKS * CH
N_PAD = 10112
WIDTH = 128


def _prologue1_body(x_ref, w_ref, b_ref, a_ref, ab_ref, out_ref):
    xn = jnp.dot(x_ref[...], w_ref[...], preferred_element_type=jnp.float32)
    xn = xn + b_ref[...]
    ar = jnp.dot(xn, a_ref[0:H, :], preferred_element_type=jnp.float32)
    u = (ar + ab_ref[...]) * xn
    z = jnp.zeros((xn.shape[0], WIDTH - 3 * H), jnp.float32)
    out_ref[...] = jnp.concatenate([u, xn, jnp.ones_like(xn), z], axis=1)


def _mid_body(acc_ref, t1_ref, a1_ref, ab1_ref, w2_ref, b2_ref, a2_ref,
              ab2_ref, t2_ref, cnt_ref):
    S = (acc_ref[0] + acc_ref[1])[0:N, :]
    S1 = S[:, 0:H]
    S2 = S[:, H:2 * H]
    deg = S[:, 2 * H:3 * H]
    xn1 = t1_ref[:, H:2 * H]
    ar1 = jnp.dot(xn1, a1_ref[0:H, :], preferred_element_type=jnp.float32)
    ac1 = jnp.dot(xn1, a1_ref[H:2 * H, :], preferred_element_type=jnp.float32)
    self1 = (ar1 + ac1 + ab1_ref[...]) * xn1
    cnt = deg + 1.0
    h = jnp.maximum((S1 + ac1 * S2 + self1) / cnt, 0.0)
    xn2 = jnp.dot(h, w2_ref[...], preferred_element_type=jnp.float32)
    xn2 = xn2 + b2_ref[...]
    ar2 = jnp.dot(xn2, a2_ref[0:C, :], preferred_element_type=jnp.float32)
    u2 = (ar2 + ab2_ref[...]) * xn2
    z = jnp.zeros((xn2.shape[0], WIDTH - 2 * C), jnp.float32)
    t2_ref[...] = jnp.concatenate([u2, xn2, z], axis=1)
    cnt_ref[...] = cnt


def _final_body(acc_ref, t2_ref, cnt_ref, a2_ref, ab2_ref, out_ref):
    S = (acc_ref[0] + acc_ref[1])[0:N, :]
    S1 = S[:, 0:C]
    S2 = S[:, C:2 * C]
    xn2 = t2_ref[:, C:2 * C]
    ar2 = jnp.dot(xn2, a2_ref[0:C, :], preferred_element_type=jnp.float32)
    ac2 = jnp.dot(xn2, a2_ref[C:2 * C, :], preferred_element_type=jnp.float32)
    self2 = (ar2 + ac2 + ab2_ref[...]) * xn2
    o = (S1 + ac2 * S2 + self2) / cnt_ref[...]
    m = jnp.max(o, axis=1, keepdims=True)
    om = o - m
    lse = jnp.log(jnp.sum(jnp.exp(om), axis=1, keepdims=True))
    out_ref[...] = om - lse


def _make_sc_pass(width):
    mesh = plsc.VectorSubcoreMesh(core_axis_name="c", subcore_axis_name="s",
                                  num_cores=NC, num_subcores=NS)
    rpt = N_PAD // NS

    @functools.partial(
        pl.kernel,
        out_type=jax.ShapeDtypeStruct((NC, N_PAD, width), jnp.float32),
        mesh=mesh,
        scratch_types=[
            pltpu.VMEM((CPW1, CH), jnp.int32),
            pltpu.VMEM((CPW1, CH), jnp.int32),
            pltpu.VMEM((CH, width), jnp.float32),
            pltpu.VMEM((CH, width), jnp.float32),
            pltpu.VMEM_SHARED((N_PAD, width), jnp.float32),
            pltpu.SemaphoreType.DMA,
            pltpu.SemaphoreType.DMA,
        ],
    )
    def sc_pass(table, rowi, coli, zeros, out, idxr, idxc, rows0, rows1,
                acc, sem0, sem1):
        c = lax.axis_index("c")
        s = lax.axis_index("s")
        nch = jnp.where(c == 0, CPW0, CPW1)
        base = jnp.where(c == 0, s * CPW0, NS * CPW0 + s * CPW1)
        pltpu.sync_copy(zeros.at[pl.ds(s * rpt, rpt)],
                        acc.at[pl.ds(s * rpt, rpt)])
        pltpu.sync_copy(rowi.at[pl.ds(base, CPW0)], idxr.at[pl.ds(0, CPW0)])

        @pl.when(c == 1)
        def _():
            pltpu.sync_copy(rowi.at[pl.ds(base + CPW0, CPW1 - CPW0)],
                            idxr.at[pl.ds(CPW0, CPW1 - CPW0)])
        pltpu.sync_copy(coli.at[pl.ds(base, CPW0)], idxc.at[pl.ds(0, CPW0)])

        @pl.when(c == 1)
        def _():
            pltpu.sync_copy(coli.at[pl.ds(base + CPW0, CPW1 - CPW0)],
                            idxc.at[pl.ds(CPW0, CPW1 - CPW0)])
        plsc.subcore_barrier()

        pltpu.async_copy(table.at[idxr.at[0]], rows0, sem0)

        def body(k, carry):
            j0 = 2 * k
            pltpu.async_copy(table.at[idxr.at[j0 + 1]], rows1, sem1)
            pltpu.make_async_copy(table.at[idxr.at[j0]], rows0, sem0).wait()
            pltpu.sync_copy(rows0, acc.at[idxc.at[j0]], add=True)

            @pl.when(j0 + 2 < nch)
            def _():
                pltpu.async_copy(table.at[idxr.at[j0 + 2]], rows0, sem0)

            pltpu.make_async_copy(
                table.at[idxr.at[j0 + 1]], rows1, sem1).wait()
            pltpu.sync_copy(rows1, acc.at[idxc.at[j0 + 1]], add=True)
            return carry

        lax.fori_loop(0, nch // 2, body, 0)
        plsc.subcore_barrier()
        pltpu.sync_copy(acc.at[pl.ds(s * rpt, rpt)],
                        out.at[c, pl.ds(s * rpt, rpt)])

    return sc_pass


_make_sc_pass = functools.lru_cache(maxsize=None)(_make_sc_pass)


@jax.jit
def kernel(x, edge_index, W1, b1, A1, Ab1, W2, b2, A2, Ab2):
    f32 = jnp.float32
    rowi = jnp.concatenate(
        [edge_index[0], jnp.zeros((E_PAD - E,), jnp.int32)]
    ).reshape(N_CHUNKS, CH)
    coli = jnp.concatenate(
        [edge_index[1], jnp.full((E_PAD - E,), N, jnp.int32)]
    ).reshape(N_CHUNKS, CH)

    table1 = pl.pallas_call(
        _prologue1_body,
        out_shape=jax.ShapeDtypeStruct((N, WIDTH), f32),
    )(x, W1, b1.reshape(1, H), A1, Ab1.reshape(1, 1))

    zeros_t = jnp.zeros((N_PAD, WIDTH), f32)
    acc1 = _make_sc_pass(WIDTH)(table1, rowi, coli, zeros_t)

    table2, cnt = pl.pallas_call(
        _mid_body,
        out_shape=(
            jax.ShapeDtypeStruct((N, WIDTH), f32),
            jax.ShapeDtypeStruct((N, C), f32),
        ),
    )(acc1, table1, A1, Ab1.reshape(1, 1), W2, b2.reshape(1, C), A2,
      Ab2.reshape(1, 1))

    acc2 = _make_sc_pass(WIDTH)(table2, rowi, coli, zeros_t)

    out = pl.pallas_call(
        _final_body,
        out_shape=jax.ShapeDtypeStruct((N, C), f32),
    )(acc2, table2, cnt, A2, Ab2.reshape(1, 1))
    return out


# speedup vs baseline: 18.0000x; 2.9048x over previous
import functools

import jax
import jax.numpy as jnp
from jax import lax
from jax.experimental import pallas as pl
from jax.experimental.pallas import tpu as pltpu
from jax.experimental.pallas import tpu_sc as plsc

N = 10000
E = 160000
F_IN = 256
H = 16
C = 16

NC = 2
NS = 16
NW = NC * NS
CH = 128
CPW0 = 16
CPW1 = 64
N_CHUNKS = NS * (CPW0 + CPW1)
E_PAD = N_CHUNKS * CH
N_PAD = 10112
W1_T = 48
W2_T = 32


def _prologue1_body(x_ref, w_ref, b_ref, a_ref, ab_ref, out_ref):
    xn = jnp.dot(x_ref[...], w_ref[...], preferred_element_type=jnp.float32)
    xn = xn + b_ref[...]
    ar = jnp.dot(xn, a_ref[0:H, :], preferred_element_type=jnp.float32)
    u = (ar + ab_ref[...]) * xn
    out_ref[...] = jnp.concatenate([u, xn, jnp.ones_like(xn)], axis=1)


def _mid_body(acc_ref, t1_ref, a1_ref, ab1_ref, w2_ref, b2_ref, a2_ref,
              ab2_ref, t2_ref, cnt_ref):
    S = (acc_ref[0] + acc_ref[1])[0:N, :]
    S1 = S[:, 0:H]
    S2 = S[:, H:2 * H]
    deg = S[:, 2 * H:3 * H]
    xn1 = t1_ref[:, H:2 * H]
    ar1 = jnp.dot(xn1, a1_ref[0:H, :], preferred_element_type=jnp.float32)
    ac1 = jnp.dot(xn1, a1_ref[H:2 * H, :], preferred_element_type=jnp.float32)
    self1 = (ar1 + ac1 + ab1_ref[...]) * xn1
    cnt = deg + 1.0
    h = jnp.maximum((S1 + ac1 * S2 + self1) / cnt, 0.0)
    xn2 = jnp.dot(h, w2_ref[...], preferred_element_type=jnp.float32)
    xn2 = xn2 + b2_ref[...]
    ar2 = jnp.dot(xn2, a2_ref[0:C, :], preferred_element_type=jnp.float32)
    u2 = (ar2 + ab2_ref[...]) * xn2
    t2_ref[...] = jnp.concatenate([u2, xn2], axis=1)
    cnt_ref[...] = cnt


def _final_body(acc_ref, t2_ref, cnt_ref, a2_ref, ab2_ref, out_ref):
    S = (acc_ref[0] + acc_ref[1])[0:N, :]
    S1 = S[:, 0:C]
    S2 = S[:, C:2 * C]
    xn2 = t2_ref[:, C:2 * C]
    ar2 = jnp.dot(xn2, a2_ref[0:C, :], preferred_element_type=jnp.float32)
    ac2 = jnp.dot(xn2, a2_ref[C:2 * C, :], preferred_element_type=jnp.float32)
    self2 = (ar2 + ac2 + ab2_ref[...]) * xn2
    o = (S1 + ac2 * S2 + self2) / cnt_ref[...]
    m = jnp.max(o, axis=1, keepdims=True)
    om = o - m
    lse = jnp.log(jnp.sum(jnp.exp(om), axis=1, keepdims=True))
    out_ref[...] = om - lse


def _make_sc_pass(width):
    mesh = plsc.VectorSubcoreMesh(core_axis_name="c", subcore_axis_name="s",
                                  num_cores=NC, num_subcores=NS)
    rpt = N_PAD // NS

    @functools.partial(
        pl.kernel,
        out_type=jax.ShapeDtypeStruct((NC, N_PAD, width), jnp.float32),
        mesh=mesh,
        compiler_params=pltpu.CompilerParams(use_tc_tiling_on_sc=False),
        scratch_types=[
            pltpu.VMEM((CPW1, CH), jnp.int32),
            pltpu.VMEM((CPW1, CH), jnp.int32),
            pltpu.VMEM((CH, width), jnp.float32),
            pltpu.VMEM((CH, width), jnp.float32),
            pltpu.VMEM_SHARED((N_PAD, width), jnp.float32),
            pltpu.VMEM_SHARED((N, width), jnp.float32),
            pltpu.SemaphoreType.DMA,
            pltpu.SemaphoreType.DMA,
        ],
    )
    def sc_pass(table, rowi, coli, zeros, out, idxr, idxc, rows0, rows1,
                acc, tbl_s, sem0, sem1):
        c = lax.axis_index("c")
        s = lax.axis_index("s")
        nch = jnp.where(c == 0, CPW0, CPW1)
        base = jnp.where(c == 0, s * CPW0, NS * CPW0 + s * CPW1)
        pltpu.sync_copy(zeros.at[pl.ds(s * rpt, rpt)],
                        acc.at[pl.ds(s * rpt, rpt)])
        tpt = N // NS
        pltpu.sync_copy(table.at[pl.ds(s * tpt, tpt)],
                        tbl_s.at[pl.ds(s * tpt, tpt)])
        pltpu.sync_copy(rowi.at[pl.ds(base, CPW0)], idxr.at[pl.ds(0, CPW0)])

        @pl.when(c == 1)
        def _():
            pltpu.sync_copy(rowi.at[pl.ds(base + CPW0, CPW1 - CPW0)],
                            idxr.at[pl.ds(CPW0, CPW1 - CPW0)])
        pltpu.sync_copy(coli.at[pl.ds(base, CPW0)], idxc.at[pl.ds(0, CPW0)])

        @pl.when(c == 1)
        def _():
            pltpu.sync_copy(coli.at[pl.ds(base + CPW0, CPW1 - CPW0)],
                            idxc.at[pl.ds(CPW0, CPW1 - CPW0)])
        plsc.subcore_barrier()

        pltpu.async_copy(tbl_s.at[idxr.at[0]], rows0, sem0)

        def body(k, carry):
            j0 = 2 * k
            pltpu.async_copy(tbl_s.at[idxr.at[j0 + 1]], rows1, sem1)
            pltpu.make_async_copy(tbl_s.at[idxr.at[j0]], rows0, sem0).wait()
            pltpu.sync_copy(rows0, acc.at[idxc.at[j0]], add=True)

            @pl.when(j0 + 2 < nch)
            def _():
                pltpu.async_copy(tbl_s.at[idxr.at[j0 + 2]], rows0, sem0)

            pltpu.make_async_copy(
                tbl_s.at[idxr.at[j0 + 1]], rows1, sem1).wait()
            pltpu.sync_copy(rows1, acc.at[idxc.at[j0 + 1]], add=True)
            return carry

        lax.fori_loop(0, nch // 2, body, 0)
        plsc.subcore_barrier()
        pltpu.sync_copy(acc.at[pl.ds(s * rpt, rpt)],
                        out.at[c, pl.ds(s * rpt, rpt)])

    return sc_pass


_make_sc_pass = functools.lru_cache(maxsize=None)(_make_sc_pass)


@jax.jit
def kernel(x, edge_index, W1, b1, A1, Ab1, W2, b2, A2, Ab2):
    f32 = jnp.float32
    rowi = jnp.concatenate(
        [edge_index[0], jnp.zeros((E_PAD - E,), jnp.int32)]
    ).reshape(N_CHUNKS, CH)
    coli = jnp.concatenate(
        [edge_index[1], jnp.full((E_PAD - E,), N, jnp.int32)]
    ).reshape(N_CHUNKS, CH)

    table1 = pl.pallas_call(
        _prologue1_body,
        out_shape=jax.ShapeDtypeStruct((N, W1_T), f32),
    )(x, W1, b1.reshape(1, H), A1, Ab1.reshape(1, 1))

    zeros1 = jnp.zeros((N_PAD, W1_T), f32)
    acc1 = _make_sc_pass(W1_T)(table1, rowi, coli, zeros1)

    table2, cnt = pl.pallas_call(
        _mid_body,
        out_shape=(
            jax.ShapeDtypeStruct((N, W2_T), f32),
            jax.ShapeDtypeStruct((N, C), f32),
        ),
    )(acc1, table1, A1, Ab1.reshape(1, 1), W2, b2.reshape(1, C), A2,
      Ab2.reshape(1, 1))

    zeros2 = jnp.zeros((N_PAD, W2_T), f32)
    acc2 = _make_sc_pass(W2_T)(table2, rowi, coli, zeros2)

    out = pl.pallas_call(
        _final_body,
        out_shape=jax.ShapeDtypeStruct((N, C), f32),
    )(acc2, table2, cnt, A2, Ab2.reshape(1, 1))
    return out
